# baseline (speedup 1.0000x reference)
import functools

import jax
import jax.numpy as jnp
from jax import lax
from jax.experimental import pallas as pl
from jax.experimental.pallas import tpu as pltpu
from jax.experimental.pallas import tpu_sc as plsc

N_VOTERS = 32
N_COLS = 1_000_000
HALF = N_VOTERS // 2
LANES = 16

NUM_CORES = 2
NUM_SUBCORES = 16
NW = NUM_CORES * NUM_SUBCORES

BLK_TC = 8192
BLK_SC = 768
SC_COLS = NW * BLK_SC
SC_TC_BLOCKS = SC_COLS // BLK_TC
N_COLS_TC = N_COLS - SC_COLS


def _sc_body(in_hbm, out_hbm, in_buf, out_buf):
    wid = lax.axis_index("s") * NUM_CORES + lax.axis_index("c")
    base = wid * BLK_SC
    pltpu.sync_copy(in_hbm.at[:, pl.ds(base, BLK_SC)], in_buf)

    def col_step(j, _):
        off = j * LANES
        acc = in_buf[0, pl.ds(off, LANES)]
        for i in range(1, N_VOTERS):
            acc = acc + in_buf[i, pl.ds(off, LANES)]
        out_buf[pl.ds(off, LANES)] = jnp.where(acc > HALF, 1, 0)
        return 0

    lax.fori_loop(0, BLK_SC // LANES, col_step, 0)
    pltpu.sync_copy(out_buf, out_hbm.at[pl.ds(base, BLK_SC)])


def _sc_vote(inputs):
    mesh = plsc.VectorSubcoreMesh(core_axis_name="c", subcore_axis_name="s")
    f = pl.kernel(
        _sc_body,
        out_type=jax.ShapeDtypeStruct((SC_COLS,), jnp.int32),
        mesh=mesh,
        scratch_types=[
            pltpu.VMEM((N_VOTERS, BLK_SC), jnp.int32),
            pltpu.VMEM((BLK_SC,), jnp.int32),
        ],
        compiler_params=pltpu.CompilerParams(use_tc_tiling_on_sc=False),
    )
    return f(inputs)


def _tc_body(in_ref, out_ref):
    s = jnp.sum(in_ref[...], axis=0)
    out_ref[...] = (s > HALF).astype(jnp.int32)


def _tc_vote(inputs):
    grid = -(-N_COLS_TC // BLK_TC)
    return pl.pallas_call(
        _tc_body,
        grid=(grid,),
        in_specs=[
            pl.BlockSpec((N_VOTERS, BLK_TC), lambda i: (0, i + SC_TC_BLOCKS))
        ],
        out_specs=pl.BlockSpec((BLK_TC,), lambda i: (i,)),
        out_shape=jax.ShapeDtypeStruct((N_COLS_TC,), jnp.int32),
    )(inputs)


@jax.jit
def kernel(inputs):
    sc_out = _sc_vote(inputs)
    tc_out = _tc_vote(inputs)
    return jnp.concatenate([sc_out, tc_out])


# speedup vs baseline: 1.0002x; 1.0002x over previous
import functools

import jax
import jax.numpy as jnp
from jax import lax
from jax.experimental import pallas as pl
from jax.experimental.pallas import tpu as pltpu
from jax.experimental.pallas import tpu_sc as plsc

N_VOTERS = 32
N_COLS = 1_000_000
HALF = N_VOTERS // 2
LANES = 16

NUM_CORES = 2
NUM_SUBCORES = 16
NW = NUM_CORES * NUM_SUBCORES

BLK_TC = 8192
BLK_SC = 768
SC_COLS = NW * BLK_SC
SC_TC_BLOCKS = SC_COLS // BLK_TC
N_COLS_TC = N_COLS - SC_COLS


def _sc_body(in_hbm, out_hbm, in_buf, out_buf):
    wid = lax.axis_index("s") * NUM_CORES + lax.axis_index("c")
    base = wid * BLK_SC
    pltpu.sync_copy(in_hbm.at[:, pl.ds(base, BLK_SC)], in_buf)

    def col_step(j, _):
        off = j * LANES
        acc = in_buf[0, pl.ds(off, LANES)]
        for i in range(1, N_VOTERS):
            acc = acc + in_buf[i, pl.ds(off, LANES)]
        out_buf[pl.ds(off, LANES)] = jnp.where(acc > HALF, 1, 0)
        return 0

    lax.fori_loop(0, BLK_SC // LANES, col_step, 0)
    pltpu.sync_copy(out_buf, out_hbm.at[pl.ds(base, BLK_SC)])


def _sc_vote(inputs):
    mesh = plsc.VectorSubcoreMesh(core_axis_name="c", subcore_axis_name="s")
    f = pl.kernel(
        _sc_body,
        out_type=jax.ShapeDtypeStruct((SC_COLS,), jnp.int32),
        mesh=mesh,
        scratch_types=[
            pltpu.VMEM((N_VOTERS, BLK_SC), jnp.int32),
            pltpu.VMEM((BLK_SC,), jnp.int32),
        ],
        compiler_params=pltpu.CompilerParams(
            use_tc_tiling_on_sc=False, skip_device_barrier=True
        ),
    )
    return f(inputs)


def _tc_body(in_ref, out_ref):
    s = jnp.sum(in_ref[...], axis=0)
    out_ref[...] = (s > HALF).astype(jnp.int32)


def _tc_vote(inputs):
    grid = -(-N_COLS_TC // BLK_TC)
    return pl.pallas_call(
        _tc_body,
        grid=(grid,),
        in_specs=[
            pl.BlockSpec((N_VOTERS, BLK_TC), lambda i: (0, i + SC_TC_BLOCKS))
        ],
        out_specs=pl.BlockSpec((BLK_TC,), lambda i: (i,)),
        out_shape=jax.ShapeDtypeStruct((N_COLS_TC,), jnp.int32),
    )(inputs)


@jax.jit
def kernel(inputs):
    sc_out = _sc_vote(inputs)
    tc_out = _tc_vote(inputs)
    return jnp.concatenate([sc_out, tc_out])


# speedup vs baseline: 24.4364x; 24.4307x over previous
import functools

import jax
import jax.numpy as jnp
from jax import lax
from jax.experimental import pallas as pl
from jax.experimental.pallas import tpu as pltpu
from jax.experimental.pallas import tpu_sc as plsc

N_VOTERS = 32
N_COLS = 1_000_000
HALF = N_VOTERS // 2
LANES = 16

NUM_CORES = 2
NUM_SUBCORES = 16
NW = NUM_CORES * NUM_SUBCORES

BLK_TC = 8192
BLK_SC = 768
SC_COLS = NW * BLK_SC
SC_TC_BLOCKS = SC_COLS // BLK_TC
N_COLS_TC = N_COLS - SC_COLS


def _sc_body(in_hbm, out_hbm, in_buf, out_buf):
    wid = lax.axis_index("s") * NUM_CORES + lax.axis_index("c")
    base = wid * BLK_SC
    pltpu.sync_copy(in_hbm.at[:, pl.ds(base, BLK_SC)], in_buf)

    def col_step(j, _):
        off = j * LANES
        acc = in_buf[0, pl.ds(off, LANES)]
        for i in range(1, N_VOTERS):
            acc = acc + in_buf[i, pl.ds(off, LANES)]
        out_buf[pl.ds(off, LANES)] = jnp.where(acc > HALF, 1, 0)
        return 0

    lax.fori_loop(0, BLK_SC // LANES, col_step, 0)
    pltpu.sync_copy(out_buf, out_hbm.at[pl.ds(base, BLK_SC)])


def _sc_vote(inputs):
    mesh = plsc.VectorSubcoreMesh(core_axis_name="c", subcore_axis_name="s")
    f = pl.kernel(
        _sc_body,
        out_type=jax.ShapeDtypeStruct((SC_COLS,), jnp.int32),
        mesh=mesh,
        scratch_types=[
            pltpu.VMEM((N_VOTERS, BLK_SC), jnp.int32),
            pltpu.VMEM((BLK_SC,), jnp.int32),
        ],
        compiler_params=pltpu.CompilerParams(use_tc_tiling_on_sc=True),
    )
    return f(inputs)


def _tc_body(in_ref, out_ref):
    s = jnp.sum(in_ref[...], axis=0)
    out_ref[...] = (s > HALF).astype(jnp.int32)


def _tc_vote(inputs):
    grid = -(-N_COLS_TC // BLK_TC)
    return pl.pallas_call(
        _tc_body,
        grid=(grid,),
        in_specs=[
            pl.BlockSpec((N_VOTERS, BLK_TC), lambda i: (0, i + SC_TC_BLOCKS))
        ],
        out_specs=pl.BlockSpec((BLK_TC,), lambda i: (i,)),
        out_shape=jax.ShapeDtypeStruct((N_COLS_TC,), jnp.int32),
    )(inputs)


@jax.jit
def kernel(inputs):
    sc_out = _sc_vote(inputs)
    tc_out = _tc_vote(inputs)
    return jnp.concatenate([sc_out, tc_out])


# speedup vs baseline: 29.4784x; 1.2063x over previous
import functools

import jax
import jax.numpy as jnp
from jax import lax
from jax.experimental import pallas as pl
from jax.experimental.pallas import tpu as pltpu
from jax.experimental.pallas import tpu_sc as plsc

N_VOTERS = 32
N_COLS = 1_000_000
HALF = N_VOTERS // 2
LANES = 16

NUM_CORES = 2
NUM_SUBCORES = 16
NW = NUM_CORES * NUM_SUBCORES

BLK_TC = 8192
BLK_SC = 3072
MAX_K = 3
SC_BLOCKS = NW * MAX_K
SC_COLS = SC_BLOCKS * BLK_SC
SC_TC_BLOCKS = SC_COLS // BLK_TC
N_COLS_TC = N_COLS - SC_COLS


def _sc_body(in_hbm, out_hbm, in_buf, out_buf):
    wid = lax.axis_index("s") * NUM_CORES + lax.axis_index("c")

    def block_step(k, _):
        base = (k * NW + wid) * BLK_SC
        pltpu.sync_copy(in_hbm.at[:, pl.ds(base, BLK_SC)], in_buf)

        def col_step(j, _):
            off = j * LANES
            acc = in_buf[0, pl.ds(off, LANES)]
            for i in range(1, N_VOTERS):
                acc = acc + in_buf[i, pl.ds(off, LANES)]
            out_buf[pl.ds(off, LANES)] = jnp.where(acc > HALF, 1, 0)
            return 0

        lax.fori_loop(0, BLK_SC // LANES, col_step, 0)
        pltpu.sync_copy(out_buf, out_hbm.at[pl.ds(base, BLK_SC)])
        return 0

    lax.fori_loop(0, MAX_K, block_step, 0)


def _sc_vote(inputs):
    mesh = plsc.VectorSubcoreMesh(core_axis_name="c", subcore_axis_name="s")
    f = pl.kernel(
        _sc_body,
        out_type=jax.ShapeDtypeStruct((SC_COLS,), jnp.int32),
        mesh=mesh,
        scratch_types=[
            pltpu.VMEM((N_VOTERS, BLK_SC), jnp.int32),
            pltpu.VMEM((BLK_SC,), jnp.int32),
        ],
        compiler_params=pltpu.CompilerParams(use_tc_tiling_on_sc=True),
    )
    return f(inputs)


def _tc_body(in_ref, out_ref):
    s = jnp.sum(in_ref[...], axis=0)
    out_ref[...] = (s > HALF).astype(jnp.int32)


def _tc_vote(inputs):
    grid = -(-N_COLS_TC // BLK_TC)
    return pl.pallas_call(
        _tc_body,
        grid=(grid,),
        in_specs=[
            pl.BlockSpec((N_VOTERS, BLK_TC), lambda i: (0, i + SC_TC_BLOCKS))
        ],
        out_specs=pl.BlockSpec((BLK_TC,), lambda i: (i,)),
        out_shape=jax.ShapeDtypeStruct((N_COLS_TC,), jnp.int32),
    )(inputs)


@jax.jit
def kernel(inputs):
    sc_out = _sc_vote(inputs)
    tc_out = _tc_vote(inputs)
    return jnp.concatenate([sc_out, tc_out])


# speedup vs baseline: 33.7855x; 1.1461x over previous
import functools

import jax
import jax.numpy as jnp
from jax import lax
from jax.experimental import pallas as pl
from jax.experimental.pallas import tpu as pltpu
from jax.experimental.pallas import tpu_sc as plsc

N_VOTERS = 32
N_COLS = 1_000_000
HALF = N_VOTERS // 2
LANES = 16

NUM_CORES = 2
NUM_SUBCORES = 16
NW = NUM_CORES * NUM_SUBCORES

BLK_TC = 16384
BLK_SC = 3584
MAX_K = 4
SC_BLOCKS = NW * MAX_K
SC_COLS = SC_BLOCKS * BLK_SC
SC_TC_BLOCKS = SC_COLS // BLK_TC
N_COLS_TC = N_COLS - SC_COLS


def _sc_body(in_hbm, out_hbm, in_buf, out_buf):
    wid = lax.axis_index("s") * NUM_CORES + lax.axis_index("c")

    def block_step(k, _):
        base = (k * NW + wid) * BLK_SC
        pltpu.sync_copy(in_hbm.at[:, pl.ds(base, BLK_SC)], in_buf)

        def col_step(j, _):
            off = j * LANES
            acc = in_buf[0, pl.ds(off, LANES)]
            for i in range(1, N_VOTERS):
                acc = acc + in_buf[i, pl.ds(off, LANES)]
            out_buf[pl.ds(off, LANES)] = jnp.where(acc > HALF, 1, 0)
            return 0

        lax.fori_loop(0, BLK_SC // LANES, col_step, 0)
        pltpu.sync_copy(out_buf, out_hbm.at[pl.ds(base, BLK_SC)])
        return 0

    lax.fori_loop(0, MAX_K, block_step, 0)


def _sc_vote(inputs):
    mesh = plsc.VectorSubcoreMesh(core_axis_name="c", subcore_axis_name="s")
    f = pl.kernel(
        _sc_body,
        out_type=jax.ShapeDtypeStruct((SC_COLS,), jnp.int32),
        mesh=mesh,
        scratch_types=[
            pltpu.VMEM((N_VOTERS, BLK_SC), jnp.int32),
            pltpu.VMEM((BLK_SC,), jnp.int32),
        ],
        compiler_params=pltpu.CompilerParams(use_tc_tiling_on_sc=True),
    )
    return f(inputs)


def _tc_body(in_ref, out_ref):
    s = jnp.sum(in_ref[...], axis=0)
    out_ref[...] = (s > HALF).astype(jnp.int32)


def _tc_vote(inputs):
    grid = -(-N_COLS_TC // BLK_TC)
    return pl.pallas_call(
        _tc_body,
        grid=(grid,),
        in_specs=[
            pl.BlockSpec((N_VOTERS, BLK_TC), lambda i: (0, i + SC_TC_BLOCKS))
        ],
        out_specs=pl.BlockSpec((BLK_TC,), lambda i: (i,)),
        out_shape=jax.ShapeDtypeStruct((N_COLS_TC,), jnp.int32),
    )(inputs)


@jax.jit
def kernel(inputs):
    sc_out = _sc_vote(inputs)
    tc_out = _tc_vote(inputs)
    return jnp.concatenate([sc_out, tc_out])


# speedup vs baseline: 35.9576x; 1.0643x over previous
import functools

import jax
import jax.numpy as jnp
from jax import lax
from jax.experimental import pallas as pl
from jax.experimental.pallas import tpu as pltpu
from jax.experimental.pallas import tpu_sc as plsc

N_VOTERS = 32
N_COLS = 1_000_000
HALF = N_VOTERS // 2
LANES = 16

NUM_CORES = 2
NUM_SUBCORES = 16
NW = NUM_CORES * NUM_SUBCORES

BLK_TC = 16384
BLK_SC = 3072
MAX_K = 4
SC_BLOCKS = NW * MAX_K
SC_COLS = SC_BLOCKS * BLK_SC
SC_TC_BLOCKS = SC_COLS // BLK_TC
N_COLS_TC = N_COLS - SC_COLS


def _sc_body(in_hbm, out_hbm, in_buf, out_buf):
    wid = lax.axis_index("s") * NUM_CORES + lax.axis_index("c")

    def block_step(k, _):
        base = (k * NW + wid) * BLK_SC
        pltpu.sync_copy(in_hbm.at[:, pl.ds(base, BLK_SC)], in_buf)

        def col_step(j, _):
            off = j * LANES
            acc = in_buf[0, pl.ds(off, LANES)]
            for i in range(1, N_VOTERS):
                acc = acc + in_buf[i, pl.ds(off, LANES)]
            out_buf[pl.ds(off, LANES)] = jnp.where(acc > HALF, 1, 0)
            return 0

        lax.fori_loop(0, BLK_SC // LANES, col_step, 0)
        pltpu.sync_copy(out_buf, out_hbm.at[pl.ds(base, BLK_SC)])
        return 0

    lax.fori_loop(0, MAX_K, block_step, 0)


def _sc_vote(inputs):
    mesh = plsc.VectorSubcoreMesh(core_axis_name="c", subcore_axis_name="s")
    f = pl.kernel(
        _sc_body,
        out_type=jax.ShapeDtypeStruct((SC_COLS,), jnp.int32),
        mesh=mesh,
        scratch_types=[
            pltpu.VMEM((N_VOTERS, BLK_SC), jnp.int32),
            pltpu.VMEM((BLK_SC,), jnp.int32),
        ],
        compiler_params=pltpu.CompilerParams(use_tc_tiling_on_sc=True),
    )
    return f(inputs)


def _tc_body(in_ref, out_ref):
    s = jnp.sum(in_ref[...], axis=0)
    out_ref[...] = (s > HALF).astype(jnp.int32)


def _tc_vote(inputs):
    grid = -(-N_COLS_TC // BLK_TC)
    return pl.pallas_call(
        _tc_body,
        grid=(grid,),
        in_specs=[
            pl.BlockSpec((N_VOTERS, BLK_TC), lambda i: (0, i + SC_TC_BLOCKS))
        ],
        out_specs=pl.BlockSpec((BLK_TC,), lambda i: (i,)),
        out_shape=jax.ShapeDtypeStruct((N_COLS_TC,), jnp.int32),
    )(inputs)


@jax.jit
def kernel(inputs):
    sc_out = _sc_vote(inputs)
    tc_out = _tc_vote(inputs)
    return jnp.concatenate([sc_out, tc_out])
